# baseline (speedup 1.0000x reference)
import functools
import math

import jax
import jax.numpy as jnp
from jax import lax
from jax.experimental import pallas as pl
from jax.experimental.pallas import tpu as pltpu
from jax.experimental.pallas import tpu_sc as plsc

_S = 256
_L = 16
_NW = 32


def _rank_body(at_ref, s_ref, rank_out, stone_out, dist_out, ang_out,
               dist_scr):
    at = at_ref[...]
    s = s_ref[...]
    bn = at.shape[1]

    ayr = at[1:2, :]
    axr = at[2:3, :]
    sy = s[:, 1:2]
    sx = s[:, 2:3]

    dy = sy - ayr
    dx = sx - axr
    d2 = dy * dy + dx * dx
    dist = jnp.sqrt(d2)
    raw = jnp.arctan2(-dy, dx) * (180.0 / math.pi)
    ang = jnp.where(raw > 0, raw, raw + 360.0)
    stone = jnp.broadcast_to(s[:, 0:1], (_S, bn))

    dist_scr[...] = dist
    mask = at[0:1, :] == 0.0

    CH = 64
    for c0 in range(0, _S, CH):
        dchunk = dist[c0:c0 + CH, :]
        elem = lax.broadcasted_iota(jnp.int32, (CH, bn), 0) + c0

        def jcol(j, acc, dchunk=dchunk, elem=elem):
            kj = dist_scr[j][None, :]
            lt = kj < dchunk
            le = kj <= dchunk
            cond = lt | (le & (elem > j))
            return acc + jnp.where(cond, 1.0, 0.0)

        acc = lax.fori_loop(0, _S, jcol, jnp.zeros((CH, bn), jnp.float32))
        rank_out[c0:c0 + CH, :] = acc.astype(jnp.int32) * 3

    stone_out[...] = jnp.where(mask, stone, 0.0)
    dist_out[...] = jnp.where(mask, dist, 0.0)
    ang_out[...] = jnp.where(mask, ang, 0.0)


def _tc_rank_call(a, s):
    n = a.shape[0]
    bn = 128
    grid = n // bn
    at = jnp.pad(a.T, ((0, 5), (0, 0)))
    out_shape = [jax.ShapeDtypeStruct((_S, n), jnp.int32)] + [
        jax.ShapeDtypeStruct((_S, n), jnp.float32)] * 3
    return pl.pallas_call(
        _rank_body,
        grid=(grid,),
        in_specs=[
            pl.BlockSpec((8, bn), lambda i: (0, i)),
            pl.BlockSpec((_S, 3), lambda i: (0, 0)),
        ],
        out_specs=[pl.BlockSpec((_S, bn), lambda i: (0, i))] * 4,
        out_shape=out_shape,
        scratch_shapes=[pltpu.VMEM((_S, bn), jnp.float32)],
    )(at, s)


def _sc_scatter_call(rank3, stone_m, dist_m, ang_m):
    n = rank3.shape[0] // _S
    rows_w = n // _NW
    groups = rows_w // _L
    mesh = plsc.VectorSubcoreMesh(core_axis_name="c", subcore_axis_name="s")

    @functools.partial(
        pl.kernel,
        out_type=jax.ShapeDtypeStruct((n * 3 * _S,), jnp.float32),
        mesh=mesh,
        scratch_types=[
            pltpu.VMEM((_L * _S,), jnp.int32),
            pltpu.VMEM((_L * _S,), jnp.float32),
            pltpu.VMEM((_L * _S,), jnp.float32),
            pltpu.VMEM((_L * _S,), jnp.float32),
            pltpu.VMEM((_L * 3 * _S,), jnp.float32),
        ],
        compiler_params=pltpu.CompilerParams(needs_layout_passes=False),
    )
    def k(rank_h, stone_h, dist_h, ang_h, out_h, rank_v, stone_v, dist_v,
          ang_v, out_v):
        wid = lax.axis_index("s") * 2 + lax.axis_index("c")

        def group_body(g, _):
            base = (wid * rows_w + g * _L) * _S
            pltpu.sync_copy(rank_h.at[pl.ds(base, _L * _S)], rank_v)
            pltpu.sync_copy(stone_h.at[pl.ds(base, _L * _S)], stone_v)
            pltpu.sync_copy(dist_h.at[pl.ds(base, _L * _S)], dist_v)
            pltpu.sync_copy(ang_h.at[pl.ds(base, _L * _S)], ang_v)

            def row_body(r, _):
                rbase = r * (3 * _S)

                def chunk_body(c, _):
                    o = r * _S + c * _L
                    r3 = rank_v[pl.ds(o, _L)] + rbase
                    plsc.store_scatter(out_v, [r3],
                                       stone_v[pl.ds(o, _L)])
                    plsc.store_scatter(out_v, [r3 + 1],
                                       dist_v[pl.ds(o, _L)])
                    plsc.store_scatter(out_v, [r3 + 2],
                                       ang_v[pl.ds(o, _L)])
                    return 0

                lax.fori_loop(0, _S // _L, chunk_body, 0, unroll=4)
                return 0

            lax.fori_loop(0, _L, row_body, 0)
            pltpu.sync_copy(out_v, out_h.at[pl.ds(base * 3, _L * 3 * _S)])
            return 0

        lax.fori_loop(0, groups, group_body, 0)

    return k(rank3, stone_m, dist_m, ang_m)


@jax.jit
def kernel(all_coord_input, stone_coord_input):
    a = all_coord_input.astype(jnp.float32)
    s = stone_coord_input.astype(jnp.float32)
    n = a.shape[0]
    rank3, stone_m, dist_m, ang_m = _tc_rank_call(a, s)
    out = _sc_scatter_call(rank3.T.reshape(-1), stone_m.T.reshape(-1),
                           dist_m.T.reshape(-1), ang_m.T.reshape(-1))
    return out.reshape(n, _S, 3)


# speedup vs baseline: 1.9765x; 1.9765x over previous
import functools
import math

import jax
import jax.numpy as jnp
from jax import lax
from jax.experimental import pallas as pl
from jax.experimental.pallas import tpu as pltpu
from jax.experimental.pallas import tpu_sc as plsc

_S = 256
_L = 16
_NW = 32

_HALF_PI = math.pi / 2.0
_PI = math.pi
_R2D = 180.0 / math.pi

_ATAN_C = (0.9999965494666837, -0.33318339140806397, 0.19814843475101232,
           -0.1325642608068135, 0.0800028446589445, -0.033907658670957394,
           0.006905941419869666)


def _rsqrt(x):
    i = plsc.bitcast(x, jnp.int32)
    i = 0x5F3759DF - (i >> 1)
    y = plsc.bitcast(i, jnp.float32)
    for _ in range(3):
        y = y * (1.5 - 0.5 * x * y * y)
    return y


def _atan2deg(n, d):
    an = jnp.abs(n)
    ad = jnp.abs(d)
    mx = jnp.maximum(an, ad)
    mn = jnp.minimum(an, ad)
    t = jnp.where(mx > 0, mn / mx, 0.0)
    t2 = t * t
    z = jnp.full((_L,), _ATAN_C[-1], jnp.float32)
    for c in reversed(_ATAN_C[:-1]):
        z = z * t2 + c
    z = z * t
    z = jnp.where(an > ad, _HALF_PI - z, z)
    z = jnp.where(d < 0, _PI - z, z)
    z = jnp.where(n < 0, -z, z)
    deg = z * _R2D
    return jnp.where(deg > 0, deg, deg + 360.0)


def _sc_sort_call(occ, ay, ax, sval, sy, sx, interpret=False):
    n = occ.shape[0]
    rows_w = n // _NW
    groups = rows_w // _L
    mesh = plsc.VectorSubcoreMesh(core_axis_name="c", subcore_axis_name="s")

    @functools.partial(
        pl.kernel,
        out_type=jax.ShapeDtypeStruct((n, 3 * _S), jnp.float32),
        mesh=mesh,
        scratch_types=[
            pltpu.VMEM((_S,), jnp.float32),
            pltpu.VMEM((_S,), jnp.float32),
            pltpu.VMEM((_S,), jnp.float32),
            pltpu.VMEM((_L,), jnp.float32),
            pltpu.VMEM((_L,), jnp.float32),
            pltpu.VMEM((_L,), jnp.float32),
            pltpu.VMEM((_S, _L), jnp.float32),
            pltpu.VMEM((_S, _L), jnp.int32),
            pltpu.VMEM((_L, 3 * _S), jnp.float32),
        ],
        interpret=interpret,
        compiler_params=pltpu.CompilerParams(needs_layout_passes=False),
    )
    def k(occ_h, ay_h, ax_h, sval_h, sy_h, sx_h, out_h,
          sval_v, sy_v, sx_v, occ_v, ay_v, ax_v, d2_t, idx_t, out_v):
        wid = lax.axis_index("s") * 2 + lax.axis_index("c")
        pltpu.sync_copy(sval_h, sval_v)
        pltpu.sync_copy(sy_h, sy_v)
        pltpu.sync_copy(sx_h, sx_v)
        lanev = lax.iota(jnp.int32, _L)

        def group_body(g, _):
            base = wid * rows_w + g * _L
            pltpu.sync_copy(occ_h.at[pl.ds(base, _L)], occ_v)
            pltpu.sync_copy(ay_h.at[pl.ds(base, _L)], ay_v)
            pltpu.sync_copy(ax_h.at[pl.ds(base, _L)], ax_v)
            ayv = ay_v[...]
            axv = ax_v[...]
            occv = occ_v[...]

            def dstone(ki, _):
                kvec = jnp.full((_L,), ki, jnp.int32)
                syk = plsc.load_gather(sy_v, [kvec])
                sxk = plsc.load_gather(sx_v, [kvec])
                dy = syk - ayv
                dx = sxk - axv
                d2_t[ki] = dy * dy + dx * dx
                idx_t[ki] = kvec
                return 0

            lax.fori_loop(0, _S, dstone, 0, unroll=8)

            for lk in range(1, 9):
                kk = 1 << lk

                def stage(t, _, kk=kk):
                    j = kk >> (1 + t)

                    def ce(i, _):
                        a = 2 * i - (i & (j - 1))
                        b = a + j
                        da = d2_t[a]
                        db = d2_t[b]
                        ia = idx_t[a]
                        ib = idx_t[b]
                        asc = ((a & kk) == 0).astype(jnp.int32)
                        lt = ((da < db) | ((da == db) & (ia < ib)))
                        sel = lt.astype(jnp.int32) == jnp.full((_L,), asc)
                        d2_t[a] = jnp.where(sel, da, db)
                        d2_t[b] = jnp.where(sel, db, da)
                        idx_t[a] = jnp.where(sel, ia, ib)
                        idx_t[b] = jnp.where(sel, ib, ia)
                        return 0

                    lax.fori_loop(0, _S // 2, ce, 0, unroll=8)
                    return 0

                lax.fori_loop(0, lk, stage, 0)

            maskv = jnp.where(occv == 0.0, 1.0, 0.0)

            def outk(ki, _):
                sidx = idx_t[ki]
                d2s = d2_t[ki]
                sv = plsc.load_gather(sval_v, [sidx])
                syk = plsc.load_gather(sy_v, [sidx])
                sxk = plsc.load_gather(sx_v, [sidx])
                dy = syk - ayv
                dx = sxk - axv
                dist = jnp.where(d2s > 0, d2s * _rsqrt(d2s), 0.0)
                ang = _atan2deg(-dy, dx)
                col = ki * 3
                plsc.store_scatter(out_v, [lanev, jnp.full((_L,), col)],
                                   sv * maskv)
                plsc.store_scatter(out_v, [lanev, jnp.full((_L,), col + 1)],
                                   dist * maskv)
                plsc.store_scatter(out_v, [lanev, jnp.full((_L,), col + 2)],
                                   ang * maskv)
                return 0

            lax.fori_loop(0, _S, outk, 0, unroll=4)
            pltpu.sync_copy(out_v, out_h.at[pl.ds(base, _L)])
            return 0

        lax.fori_loop(0, groups, group_body, 0)

    return k(occ, ay, ax, sval, sy, sx)


@jax.jit
def kernel(all_coord_input, stone_coord_input):
    a = all_coord_input.astype(jnp.float32)
    s = stone_coord_input.astype(jnp.float32)
    n = a.shape[0]
    out = _sc_sort_call(a[:, 0], a[:, 1], a[:, 2],
                        s[:, 0], s[:, 1], s[:, 2])
    return out.reshape(n, _S, 3)
